# baseline (speedup 1.0000x reference)
import functools

import jax
import jax.numpy as jnp
from jax import lax
from jax.experimental import pallas as pl
from jax.experimental.pallas import tpu as pltpu

ND = 1024
BN = 1024
BIG_F = 3.0e38
BIG_I = 2**31 - 1


def _body(xyz_ref, oxyzm2_ref, feat_ref, ofeat_ref, w0_ref, w1_ref, out_ref):
    k = xyz_ref[0]
    um2 = oxyzm2_ref[0]

    dotp = lax.dot_general(k, um2, (((1,), (1,)), ((), ())),
                           precision=lax.Precision.HIGHEST,
                           preferred_element_type=jnp.float32)
    kn = jnp.sum(k * k, axis=1, keepdims=True)
    un = 0.25 * jnp.sum(um2 * um2, axis=1, keepdims=True)
    d2 = jnp.maximum(dotp + kn + un.reshape(1, BN), 0.0)

    iota0 = lax.broadcasted_iota(jnp.int32, (ND, BN), 0)

    keyb = lax.bitcast_convert_type(d2, jnp.int32)
    key = lax.bitcast_convert_type(keyb | iota0, jnp.float32)

    kms = []
    hits = []
    for j in range(3):
        km = jnp.min(key, axis=0, keepdims=True)
        hit = key == km
        kms.append(km)
        hits.append(hit)
        if j < 2:
            key = jnp.where(hit, BIG_F, key)

    d2s = kms
    r0 = 1.0 / (d2s[0] + 1e-8)
    r1 = 1.0 / (d2s[1] + 1e-8)
    r2 = 1.0 / (d2s[2] + 1e-8)
    rnorm = 1.0 / (r0 + r1 + r2)

    st = jnp.where(hits[0], r0 * rnorm, 0.0)
    st = jnp.where(hits[1], r1 * rnorm, st)
    st = jnp.where(hits[2], r2 * rnorm, st)
    st = st.astype(jnp.bfloat16)

    feat = feat_ref[0]
    interp = lax.dot_general(feat, st, (((1,), (0,)), ((), ())),
                             preferred_element_type=jnp.float32)

    x = jnp.concatenate([interp.astype(jnp.bfloat16), ofeat_ref[0]], axis=0)
    h = lax.dot_general(w0_ref[...], x, (((1,), (0,)), ((), ())),
                        preferred_element_type=jnp.float32)
    h = jnp.maximum(h, 0.0).astype(jnp.bfloat16)
    h = lax.dot_general(w1_ref[...], h, (((1,), (0,)), ((), ())),
                        preferred_element_type=jnp.float32)
    out_ref[0] = jnp.maximum(h, 0.0)


@functools.partial(jax.jit, static_argnames=("interpret",))
def _run(xyzs, original_xyzs, features, original_features, W0, W1,
         interpret=False):
    B, T, ND_, _ = xyzs.shape
    NO = original_xyzs.shape[2]
    CIN = features.shape[2]
    CORIG = original_features.shape[2]
    MLP1 = W1.shape[0]
    BT = B * T
    nob = NO // BN

    xyz_f = xyzs.reshape(BT, ND_, 3)
    oxyz_m2 = (-2.0 * original_xyzs).reshape(BT, NO, 3)
    feat_f = features.reshape(BT, CIN, ND_).astype(jnp.bfloat16)
    ofeat_f = original_features.reshape(BT, CORIG, NO).astype(jnp.bfloat16)
    W0 = W0.astype(jnp.bfloat16)
    W1 = W1.astype(jnp.bfloat16)

    out = pl.pallas_call(
        _body,
        grid=(BT, nob),
        in_specs=[
            pl.BlockSpec((1, ND_, 3), lambda f, n: (f, 0, 0)),
            pl.BlockSpec((1, BN, 3), lambda f, n: (f, n, 0)),
            pl.BlockSpec((1, CIN, ND_), lambda f, n: (f, 0, 0)),
            pl.BlockSpec((1, CORIG, BN), lambda f, n: (f, 0, n)),
            pl.BlockSpec((W0.shape[0], W0.shape[1]), lambda f, n: (0, 0)),
            pl.BlockSpec((MLP1, W1.shape[1]), lambda f, n: (0, 0)),
        ],
        out_specs=pl.BlockSpec((1, MLP1, BN), lambda f, n: (f, 0, n)),
        out_shape=jax.ShapeDtypeStruct((BT, MLP1, NO), jnp.float32),
        compiler_params=pltpu.CompilerParams(
            dimension_semantics=("arbitrary", "arbitrary"),
        ),
        interpret=interpret,
    )(xyz_f, oxyz_m2, feat_f, ofeat_f, W0, W1)

    return original_xyzs, out.reshape(B, T, MLP1, NO)


def kernel(xyzs, original_xyzs, features, original_features, W0, W1):
    return _run(xyzs, original_xyzs, features, original_features, W0, W1)


# speedup vs baseline: 1.3892x; 1.3892x over previous
import functools

import jax
import jax.numpy as jnp
from jax import lax
from jax.experimental import pallas as pl
from jax.experimental.pallas import tpu as pltpu

ND = 1024
BN = 1024
BIG_F = 3.0e38
BIG_I = 2**31 - 1


def _body(xyz_ref, oxyzt_ref, feat_ref, ofeat_ref, w0_ref, w1_ref, out_ref):
    k = xyz_ref[0]
    ut = oxyzt_ref[0]

    d2 = None
    for c in range(3):
        diff = k[:, c:c + 1] - ut[c:c + 1, :]
        d2 = diff * diff if d2 is None else d2 + diff * diff

    iota0 = lax.broadcasted_iota(jnp.int32, (ND, BN), 0)

    keyb = lax.bitcast_convert_type(d2, jnp.int32)
    key = lax.bitcast_convert_type(keyb | iota0, jnp.float32)

    kms = []
    hits = []
    for j in range(3):
        km = jnp.min(key, axis=0, keepdims=True)
        hit = key == km
        kms.append(km)
        hits.append(hit)
        if j < 2:
            key = jnp.where(hit, BIG_F, key)

    d2s = kms
    r0 = 1.0 / (d2s[0] + 1e-8)
    r1 = 1.0 / (d2s[1] + 1e-8)
    r2 = 1.0 / (d2s[2] + 1e-8)
    rnorm = 1.0 / (r0 + r1 + r2)

    st = jnp.where(hits[0], r0 * rnorm, 0.0)
    st = jnp.where(hits[1], r1 * rnorm, st)
    st = jnp.where(hits[2], r2 * rnorm, st)
    st = st.astype(jnp.bfloat16)

    feat = feat_ref[0]
    interp = lax.dot_general(feat, st, (((1,), (0,)), ((), ())),
                             preferred_element_type=jnp.float32)

    x = jnp.concatenate([interp.astype(jnp.bfloat16), ofeat_ref[0]], axis=0)
    h = lax.dot_general(w0_ref[...], x, (((1,), (0,)), ((), ())),
                        preferred_element_type=jnp.float32)
    h = jnp.maximum(h, 0.0).astype(jnp.bfloat16)
    h = lax.dot_general(w1_ref[...], h, (((1,), (0,)), ((), ())),
                        preferred_element_type=jnp.float32)
    out_ref[0] = jnp.maximum(h, 0.0)


@functools.partial(jax.jit, static_argnames=("interpret",))
def _run(xyzs, original_xyzs, features, original_features, W0, W1,
         interpret=False):
    B, T, ND_, _ = xyzs.shape
    NO = original_xyzs.shape[2]
    CIN = features.shape[2]
    CORIG = original_features.shape[2]
    MLP1 = W1.shape[0]
    BT = B * T
    nob = NO // BN

    xyz_f = xyzs.reshape(BT, ND_, 3)
    oxyz_t = original_xyzs.reshape(BT, NO, 3).transpose(0, 2, 1)
    feat_f = features.reshape(BT, CIN, ND_).astype(jnp.bfloat16)
    ofeat_f = original_features.reshape(BT, CORIG, NO).astype(jnp.bfloat16)
    W0 = W0.astype(jnp.bfloat16)
    W1 = W1.astype(jnp.bfloat16)

    out = pl.pallas_call(
        _body,
        grid=(BT, nob),
        in_specs=[
            pl.BlockSpec((1, ND_, 3), lambda f, n: (f, 0, 0)),
            pl.BlockSpec((1, 3, BN), lambda f, n: (f, 0, n)),
            pl.BlockSpec((1, CIN, ND_), lambda f, n: (f, 0, 0)),
            pl.BlockSpec((1, CORIG, BN), lambda f, n: (f, 0, n)),
            pl.BlockSpec((W0.shape[0], W0.shape[1]), lambda f, n: (0, 0)),
            pl.BlockSpec((MLP1, W1.shape[1]), lambda f, n: (0, 0)),
        ],
        out_specs=pl.BlockSpec((1, MLP1, BN), lambda f, n: (f, 0, n)),
        out_shape=jax.ShapeDtypeStruct((BT, MLP1, NO), jnp.float32),
        compiler_params=pltpu.CompilerParams(
            dimension_semantics=("arbitrary", "arbitrary"),
        ),
        interpret=interpret,
    )(xyz_f, oxyz_t, feat_f, ofeat_f, W0, W1)

    return original_xyzs, out.reshape(B, T, MLP1, NO)


def kernel(xyzs, original_xyzs, features, original_features, W0, W1):
    return _run(xyzs, original_xyzs, features, original_features, W0, W1)
